# baseline (speedup 1.0000x reference)
import functools

import jax
import jax.numpy as jnp
from jax import lax
from jax.experimental import pallas as pl
from jax.experimental.pallas import tpu as pltpu
from jax.experimental.pallas import tpu_sc as plsc

_N = 10000
_E = 320000
_D = 128
_G = 64
_EPS_BN = 1e-5

_NC = 2
_NS = 16
_NW = _NC * _NS
_K = 128
_NB = 4
_KG = _K * _NB
_NPW = 320
_NOUT = _NW * _NPW
_ACCR = _NPW + 8

_sc_mesh = plsc.VectorSubcoreMesh(core_axis_name="c", subcore_axis_name="s")


@functools.partial(
    pl.kernel,
    mesh=_sc_mesh,
    out_type=jax.ShapeDtypeStruct((_NOUT, _D), jnp.float32),
    scratch_types=[
        pltpu.VMEM((48,), jnp.int32),
        pltpu.VMEM((_KG,), jnp.int32),
        pltpu.VMEM((_KG,), jnp.int32),
        pltpu.VMEM((_KG,), jnp.int32),
        pltpu.VMEM((_NB, _K, _D), jnp.float32),
        pltpu.SemaphoreType.DMA,
        pltpu.SemaphoreType.DMA,
        pltpu.SemaphoreType.DMA,
        pltpu.SemaphoreType.DMA,
        pltpu.VMEM_SHARED((_NS * _ACCR, _D), jnp.float32),
    ],
)
def _edge_aggregate_sc(h_hbm, src_hbm, dst_hbm, bnd_hbm, zeros_hbm, out_hbm,
                       bnd_v, sidx_v, dbuf_v, lidx_v, rows_v,
                       sem0, sem1, sem2, sem3, acc_sh):
    sems = [sem0, sem1, sem2, sem3]
    c = lax.axis_index("c")
    s = lax.axis_index("s")
    wid = c * _NS + s
    nbase = wid * _NPW
    tilebase = s * _ACCR

    pltpu.sync_copy(zeros_hbm, acc_sh.at[pl.ds(s * _ACCR, _ACCR)])
    pltpu.sync_copy(bnd_hbm, bnd_v)

    bv = bnd_v[pl.ds(wid, 16)]
    lo_raw = bv[0]
    hi_raw = bv[1]
    lo = (lo_raw // _KG) * _KG
    nch = (hi_raw - lo + (_KG - 1)) // _KG

    def body(j, carry):
        gbase = lo + j * _KG
        pltpu.sync_copy(src_hbm.at[pl.ds(gbase, _KG)], sidx_v)
        pltpu.sync_copy(dst_hbm.at[pl.ds(gbase, _KG)], dbuf_v)
        for t in range(_KG // 16):
            v = dbuf_v[pl.ds(t * 16, 16)]
            lv = v - nbase
            ok = (lv >= 0) & (lv < _NPW)
            lv = jnp.where(ok, lv, _NPW) + tilebase
            lidx_v[pl.ds(t * 16, 16)] = lv
        for b in range(_NB):
            pltpu.async_copy(h_hbm.at[sidx_v.at[pl.ds(b * _K, _K)]],
                             rows_v.at[b], sems[b])
        for b in range(_NB):
            pltpu.make_async_copy(
                h_hbm.at[sidx_v.at[pl.ds(b * _K, _K)]], rows_v.at[b],
                sems[b]).wait()
            pltpu.sync_copy(rows_v.at[b],
                            acc_sh.at[lidx_v.at[pl.ds(b * _K, _K)]], add=True)
        return carry

    lax.fori_loop(0, nch, body, 0)

    pltpu.sync_copy(acc_sh.at[pl.ds(s * _ACCR, _NPW)],
                    out_hbm.at[pl.ds(wid * _NPW, _NPW)])


def _mlp_tc(h_ref, agg_ref, w1_ref, b1_ref, g_ref, be_ref, w2_ref, b2_ref,
            o_ref):
    u = h_ref[...] + agg_ref[:_N]
    t = jnp.dot(u, w1_ref[...], preferred_element_type=jnp.float32) + b1_ref[...]
    t = jnp.maximum(t, 0.0)
    mu = jnp.mean(t, axis=0, keepdims=True)
    var = jnp.mean(jnp.square(t - mu), axis=0, keepdims=True)
    t = (t - mu) * (g_ref[...] * lax.rsqrt(var + _EPS_BN)) + be_ref[...]
    t = jnp.dot(t, w2_ref[...], preferred_element_type=jnp.float32) + b2_ref[...]
    o_ref[...] = jnp.maximum(t, 0.0)


def _pool_tc(h_ref, b_ref, o_ref):
    gids = lax.broadcasted_iota(jnp.int32, (1, _G), 1)
    onehot = (b_ref[...] == gids).astype(jnp.float32)
    o_ref[...] = lax.dot_general(
        onehot, h_ref[...], (((0,), (0,)), ((), ())),
        precision=lax.Precision.HIGHEST, preferred_element_type=jnp.float32)


def kernel(x, edge_index, batch, params):
    src = edge_index[0]
    dst = edge_index[1]
    perm = jnp.argsort(dst, stable=True)
    src_s = src[perm]
    dst_s = dst[perm]
    grid = _NPW * jnp.arange(33, dtype=jnp.int32)
    bnd = jnp.searchsorted(dst_s, grid, side="left").astype(jnp.int32)
    bnd = jnp.concatenate([bnd, jnp.full((15,), _E, jnp.int32)])
    zeros_blk = jnp.zeros((_ACCR, _D), jnp.float32)
    batch2 = batch.reshape(_N, 1)

    mlp = pl.pallas_call(
        _mlp_tc, out_shape=jax.ShapeDtypeStruct((_N, _D), jnp.float32))
    pool = pl.pallas_call(
        _pool_tc, out_shape=jax.ShapeDtypeStruct((_G, _D), jnp.float32))

    h = x
    for p in params:
        agg = _edge_aggregate_sc(h, src_s, dst_s, bnd, zeros_blk)
        h = mlp(h, agg, p["W1"], p["b1"].reshape(1, _D),
                p["gamma"].reshape(1, _D), p["beta"].reshape(1, _D),
                p["W2"], p["b2"].reshape(1, _D))
    return pool(h, batch2)


# speedup vs baseline: 1.0618x; 1.0618x over previous
import functools

import jax
import jax.numpy as jnp
from jax import lax
from jax.experimental import pallas as pl
from jax.experimental.pallas import tpu as pltpu
from jax.experimental.pallas import tpu_sc as plsc

_N = 10000
_E = 320000
_D = 128
_G = 64
_EPS_BN = 1e-5

_NC = 2
_NS = 16
_NW = _NC * _NS
_K = 128
_NB = 4
_KG = _K * _NB
_NPW = 320
_NOUT = _NW * _NPW
_ACCR = _NPW + 8

_sc_mesh = plsc.VectorSubcoreMesh(core_axis_name="c", subcore_axis_name="s")


@functools.partial(
    pl.kernel,
    mesh=_sc_mesh,
    out_type=jax.ShapeDtypeStruct((_NOUT, _D), jnp.float32),
    scratch_types=[
        pltpu.VMEM((48,), jnp.int32),
        pltpu.VMEM((_NB, _K), jnp.int32),
        pltpu.VMEM((_NB, _K), jnp.int32),
        pltpu.VMEM((_NB, _K), jnp.int32),
        pltpu.VMEM((_NB, _K, _D), jnp.float32),
        pltpu.SemaphoreType.DMA,
        pltpu.SemaphoreType.DMA,
        pltpu.SemaphoreType.DMA,
        pltpu.SemaphoreType.DMA,
        pltpu.VMEM_SHARED((_NS * _ACCR, _D), jnp.float32),
    ],
)
def _edge_aggregate_sc(h_hbm, src_hbm, dst_hbm, bnd_hbm, zeros_hbm, out_hbm,
                       bnd_v, sidx_v, dbuf_v, lidx_v, rows_v,
                       sem0, sem1, sem2, sem3, acc_sh):
    sems = [sem0, sem1, sem2, sem3]
    c = lax.axis_index("c")
    s = lax.axis_index("s")
    wid = c * _NS + s
    nbase = wid * _NPW
    tilebase = s * _ACCR

    pltpu.sync_copy(zeros_hbm, acc_sh.at[pl.ds(s * _ACCR, _ACCR)])
    pltpu.sync_copy(bnd_hbm, bnd_v)

    bv = bnd_v[pl.ds(wid, 16)]
    lo_raw = bv[0]
    hi_raw = bv[1]
    lo = (lo_raw // _KG) * _KG
    nch = (hi_raw - lo + (_KG - 1)) // _KG

    def body(j, carry):
        gbase = lo + j * _KG
        for b in range(_NB):
            base = gbase + b * _K
            pltpu.sync_copy(src_hbm.at[pl.ds(base, _K)], sidx_v.at[b])
            pltpu.sync_copy(dst_hbm.at[pl.ds(base, _K)], dbuf_v.at[b])
            for t in range(_K // 16):
                v = dbuf_v[b, pl.ds(t * 16, 16)]
                lv = v - nbase
                ok = (lv >= 0) & (lv < _NPW)
                lv = jnp.where(ok, lv, _NPW) + tilebase
                lidx_v[b, pl.ds(t * 16, 16)] = lv
            pltpu.async_copy(h_hbm.at[sidx_v.at[b]], rows_v.at[b], sems[b])
        for b in range(_NB):
            pltpu.make_async_copy(
                h_hbm.at[sidx_v.at[b]], rows_v.at[b], sems[b]).wait()
            pltpu.sync_copy(rows_v.at[b], acc_sh.at[lidx_v.at[b]], add=True)
        return carry

    lax.fori_loop(0, nch, body, 0)

    pltpu.sync_copy(acc_sh.at[pl.ds(s * _ACCR, _NPW)],
                    out_hbm.at[pl.ds(wid * _NPW, _NPW)])


def _mlp_tc(h_ref, agg_ref, w1_ref, b1_ref, g_ref, be_ref, w2_ref, b2_ref,
            o_ref):
    u = h_ref[...] + agg_ref[:_N]
    t = jnp.dot(u, w1_ref[...], preferred_element_type=jnp.float32) + b1_ref[...]
    t = jnp.maximum(t, 0.0)
    mu = jnp.mean(t, axis=0, keepdims=True)
    var = jnp.mean(jnp.square(t - mu), axis=0, keepdims=True)
    t = (t - mu) * (g_ref[...] * lax.rsqrt(var + _EPS_BN)) + be_ref[...]
    t = jnp.dot(t, w2_ref[...], preferred_element_type=jnp.float32) + b2_ref[...]
    o_ref[...] = jnp.maximum(t, 0.0)


def _pool_tc(h_ref, b_ref, o_ref):
    gids = lax.broadcasted_iota(jnp.int32, (1, _G), 1)
    onehot = (b_ref[...] == gids).astype(jnp.float32)
    o_ref[...] = lax.dot_general(
        onehot, h_ref[...], (((0,), (0,)), ((), ())),
        precision=lax.Precision.HIGHEST, preferred_element_type=jnp.float32)


def kernel(x, edge_index, batch, params):
    src = edge_index[0]
    dst = edge_index[1]
    perm = jnp.argsort(dst, stable=True)
    src_s = src[perm]
    dst_s = dst[perm]
    grid = _NPW * jnp.arange(33, dtype=jnp.int32)
    bnd = jnp.searchsorted(dst_s, grid, side="left").astype(jnp.int32)
    bnd = jnp.concatenate([bnd, jnp.full((15,), _E, jnp.int32)])
    zeros_blk = jnp.zeros((_ACCR, _D), jnp.float32)
    batch2 = batch.reshape(_N, 1)

    mlp = pl.pallas_call(
        _mlp_tc, out_shape=jax.ShapeDtypeStruct((_N, _D), jnp.float32))
    pool = pl.pallas_call(
        _pool_tc, out_shape=jax.ShapeDtypeStruct((_G, _D), jnp.float32))

    h = x
    for p in params:
        agg = _edge_aggregate_sc(h, src_s, dst_s, bnd, zeros_blk)
        h = mlp(h, agg, p["W1"], p["b1"].reshape(1, _D),
                p["gamma"].reshape(1, _D), p["beta"].reshape(1, _D),
                p["W2"], p["b2"].reshape(1, _D))
    return pool(h, batch2)
